# baseline (speedup 1.0000x reference)
import functools

import jax
import jax.numpy as jnp
from jax import lax
from jax.experimental import pallas as pl
from jax.experimental.pallas import tpu as pltpu
from jax.experimental.pallas import tpu_sc as plsc

DIM = 32
LANE = 128
NW = 32
B = 16384 * 50
B_PER_W = B // NW
CHUNK = 512
IDX_ROWS = CHUNK // LANE
N_CHUNK = B_PER_W // CHUNK
N_PAIR = N_CHUNK // 2
ROWS_PER_W = B_PER_W // LANE

_mesh = plsc.VectorSubcoreMesh(core_axis_name="c", subcore_axis_name="s")


@functools.partial(
    pl.kernel,
    mesh=_mesh,
    out_type=jax.ShapeDtypeStruct((B, DIM), jnp.float32),
    scratch_types=[
        pltpu.VMEM((IDX_ROWS, LANE), jnp.int32),
        pltpu.VMEM((IDX_ROWS, LANE), jnp.int32),
        pltpu.VMEM((CHUNK, DIM), jnp.float32),
        pltpu.VMEM((CHUNK, DIM), jnp.float32),
        pltpu.SemaphoreType.DMA,
        pltpu.SemaphoreType.DMA,
        pltpu.SemaphoreType.DMA,
    ],
    compiler_params=pltpu.CompilerParams(use_tc_tiling_on_sc=False),
)
def _gather(x_hbm, table_hbm, out_hbm, idx0, idx1, rows0, rows1,
            idx_sem, gat_sem, out_sem):
    wid = lax.axis_index("s") * 2 + lax.axis_index("c")
    row0 = wid * ROWS_PER_W

    def idx_start(c, ibuf):
        pltpu.async_copy(
            x_hbm.at[pl.ds(row0 + c * IDX_ROWS, IDX_ROWS)], ibuf, idx_sem)

    def idx_wait(ibuf):
        pltpu.make_async_copy(
            x_hbm.at[pl.ds(0, IDX_ROWS)], ibuf, idx_sem).wait()

    def out_start(c, rbuf):
        pltpu.async_copy(
            rbuf, out_hbm.at[pl.ds((row0 + c * IDX_ROWS) * LANE, CHUNK)],
            out_sem)

    def out_wait(rbuf):
        pltpu.make_async_copy(
            rbuf, out_hbm.at[pl.ds(0, CHUNK)], out_sem).wait()

    def chunk_work(c, ibuf, rbuf):
        idx_wait(ibuf)
        handles = [
            pltpu.async_copy(
                table_hbm.at[ibuf.at[j]],
                rbuf.at[pl.ds(j * LANE, LANE)],
                gat_sem,
            )
            for j in range(IDX_ROWS)
        ]
        for h in handles:
            h.wait()

        @pl.when(c + 2 < N_CHUNK)
        def _():
            idx_start(c + 2, ibuf)

        out_start(c, rbuf)

    idx_start(0, idx0)
    idx_start(1, idx1)

    def pair(g, carry):
        @pl.when(g > 0)
        def _():
            out_wait(rows0)
            out_wait(rows1)

        chunk_work(2 * g, idx0, rows0)
        chunk_work(2 * g + 1, idx1, rows1)
        return carry

    lax.fori_loop(0, N_PAIR, pair, 0)
    out_wait(rows0)
    out_wait(rows1)


def kernel(x, table):
    flat = x.reshape(-1).astype(jnp.int32).reshape(B // LANE, LANE)
    out = _gather(flat, table)
    return out.reshape(x.shape + (DIM,))
